# Initial kernel scaffold; baseline (speedup 1.0000x reference)
import jax
import jax.numpy as jnp
from jax.experimental import pallas as pl


def kernel(x, table):
    raise NotImplementedError("write your pallas kernel here")


# speedup vs baseline: 1.5693x; 1.5693x over previous
import functools

import jax
import jax.numpy as jnp
from jax import lax
from jax.experimental import pallas as pl
from jax.experimental.pallas import tpu as pltpu
from jax.experimental.pallas import tpu_sc as plsc

_EMB = 32
_B = 16384
_F = 26
_N = _B * _F
_NC = 2
_NS = 16
_NW = _NC * _NS
_NPW = _N // _NW
_CHUNK = 128
_NCHUNK = _NPW // _CHUNK
_NBUF = 4


def _body(x_hbm, table_hbm, out_hbm, idx_v, rows_v, *sems):
    wid = lax.axis_index("s") * _NC + lax.axis_index("c")
    base = wid * _NPW

    pltpu.sync_copy(x_hbm.at[wid], idx_v)

    def start_gather(g, b):
        pltpu.async_copy(table_hbm.at[idx_v.at[g]], rows_v.at[b], sems[b])

    def wait_gather(g, b):
        pltpu.make_async_copy(
            table_hbm.at[idx_v.at[g]], rows_v.at[b], sems[b]
        ).wait()

    def store(g, b):
        pltpu.sync_copy(
            rows_v.at[b], out_hbm.at[pl.ds(base + g * _CHUNK, _CHUNK)]
        )

    for b in range(_NBUF):
        start_gather(b, b)

    @pl.loop(0, _NCHUNK // _NBUF - 1)
    def _(t):
        g0 = t * _NBUF
        for b in range(_NBUF):
            g = g0 + b
            wait_gather(g, b)
            store(g, b)
            start_gather(g + _NBUF, b)

    for b in range(_NBUF):
        g = _NCHUNK - _NBUF + b
        wait_gather(g, b)
        store(g, b)


_mesh = plsc.VectorSubcoreMesh(
    core_axis_name="c", subcore_axis_name="s", num_cores=_NC, num_subcores=_NS
)

_emb = functools.partial(
    pl.kernel,
    out_type=jax.ShapeDtypeStruct((_N, _EMB), jnp.float32),
    mesh=_mesh,
    scratch_types=[
        pltpu.VMEM((_NCHUNK, _CHUNK), jnp.int32),
        pltpu.VMEM((_NBUF, _CHUNK, _EMB), jnp.float32),
    ]
    + [pltpu.SemaphoreType.DMA] * _NBUF,
    compiler_params=pltpu.CompilerParams(use_tc_tiling_on_sc=False),
)(_body)


@jax.jit
def kernel(x, table):
    xw = x.reshape(_NW, _NCHUNK, _CHUNK)
    out = _emb(xw, table)
    return out.reshape(_B, _F, _EMB)


# speedup vs baseline: 1.5764x; 1.0045x over previous
import functools

import jax
import jax.numpy as jnp
from jax import lax
from jax.experimental import pallas as pl
from jax.experimental.pallas import tpu as pltpu
from jax.experimental.pallas import tpu_sc as plsc

_EMB = 32
_B = 16384
_F = 26
_N = _B * _F
_NC = 2
_NS = 16
_NW = _NC * _NS
_NPW = _N // _NW
_CHUNK = 832
_NCHUNK = _NPW // _CHUNK
_NBUF = 4


def _body(x_hbm, table_hbm, out_hbm, idx_v, rows_v, *sems):
    wid = lax.axis_index("s") * _NC + lax.axis_index("c")
    base = wid * _NPW

    pltpu.sync_copy(x_hbm.at[wid], idx_v)

    def start_gather(g, b):
        pltpu.async_copy(table_hbm.at[idx_v.at[g]], rows_v.at[b], sems[b])

    def wait_gather(g, b):
        pltpu.make_async_copy(
            table_hbm.at[idx_v.at[g]], rows_v.at[b], sems[b]
        ).wait()

    def store(g, b):
        pltpu.sync_copy(
            rows_v.at[b], out_hbm.at[pl.ds(base + g * _CHUNK, _CHUNK)]
        )

    for b in range(_NBUF):
        start_gather(b, b)

    @pl.loop(0, _NCHUNK // _NBUF - 1)
    def _(t):
        g0 = t * _NBUF
        for b in range(_NBUF):
            g = g0 + b
            wait_gather(g, b)
            store(g, b)
            start_gather(g + _NBUF, b)

    for b in range(_NBUF):
        g = _NCHUNK - _NBUF + b
        wait_gather(g, b)
        store(g, b)


_mesh = plsc.VectorSubcoreMesh(
    core_axis_name="c", subcore_axis_name="s", num_cores=_NC, num_subcores=_NS
)

_emb = functools.partial(
    pl.kernel,
    out_type=jax.ShapeDtypeStruct((_N, _EMB), jnp.float32),
    mesh=_mesh,
    scratch_types=[
        pltpu.VMEM((_NCHUNK, _CHUNK), jnp.int32),
        pltpu.VMEM((_NBUF, _CHUNK, _EMB), jnp.float32),
    ]
    + [pltpu.SemaphoreType.DMA] * _NBUF,
    compiler_params=pltpu.CompilerParams(use_tc_tiling_on_sc=False),
)(_body)


@jax.jit
def kernel(x, table):
    xw = x.reshape(_NW, _NCHUNK, _CHUNK)
    out = _emb(xw, table)
    return out.reshape(_B, _F, _EMB)
